# baseline (speedup 1.0000x reference)
import jax
import jax.numpy as jnp
from jax.experimental import pallas as pl
from jax.experimental.pallas import tpu as pltpu

B, T, I, H, K = 512, 100, 128, 256, 8
CH = 16
G = (T + 3 + CH - 1) // CH + 1
LAST_XBLK = (T - 1) // CH


def _sigmoid(x):
    return 0.5 + 0.5 * jnp.tanh(0.5 * x)


def _gru_cell(x_bf, h_prev, wih, whh, brz, bin_, bhn, hd):
    gi = jnp.dot(x_bf, wih, preferred_element_type=jnp.float32)
    gh = jnp.dot(h_prev.astype(jnp.bfloat16), whh,
                 preferred_element_type=jnp.float32)
    rz = _sigmoid(gi[:, : 2 * hd] + gh[:, : 2 * hd] + brz)
    r = rz[:, :hd]
    zg = rz[:, hd:]
    n = jnp.tanh(gi[:, 2 * hd:] + bin_ + r * (gh[:, 2 * hd:] + bhn))
    return n + zg * (h_prev - n)


def _fused_kernel(x_ref, c_ref,
                  rw0, rw1, rw2, rw3, rw4, rw5, rw6, rw7,
                  bih0_ref, bhh0_ref, bih1_ref, bhh1_ref,
                  bih2_ref, bhh2_ref, bih3_ref, bhh3_ref,
                  xbar_ref, z_ref, q_ref,
                  h1, h2, h3, h4, s1, s2, s3, stage,
                  w0, w1, w2, w3, w4, w5, w6, w7,
                  brz0, bn0, brz1, bn1, brz2, bn2, brz3, bn3):
    g = pl.program_id(0)

    @pl.when(g == 0)
    def _init():
        h1[...] = jnp.zeros_like(h1)
        h2[...] = jnp.zeros_like(h2)
        h3[...] = jnp.zeros_like(h3)
        h4[...] = jnp.zeros_like(h4)
        s1[...] = jnp.zeros_like(s1)
        s2[...] = jnp.zeros_like(s2)
        s3[...] = jnp.zeros_like(s3)
        for dst, src in ((w0, rw0), (w1, rw1), (w2, rw2), (w3, rw3),
                         (w4, rw4), (w5, rw5), (w6, rw6), (w7, rw7)):
            dst[...] = jnp.swapaxes(src[...], 0, 1).astype(jnp.bfloat16)
        for brz, bn, bi, bh, hd in ((brz0, bn0, bih0_ref, bhh0_ref, H),
                                    (brz1, bn1, bih1_ref, bhh1_ref, H),
                                    (brz2, bn2, bih2_ref, bhh2_ref, I),
                                    (brz3, bn3, bih3_ref, bhh3_ref, I)):
            brz[...] = (bi[: 2 * hd] + bh[: 2 * hd]).reshape(1, 2 * hd)
            bn[...] = jnp.concatenate(
                [bi[2 * hd:].reshape(1, hd), bh[2 * hd:].reshape(1, hd)],
                axis=0)

    for p in range(CH - 3):
        xbar_ref[:, p, :] = stage[p, :, :]

    for j in range(CH):
        nh1 = _gru_cell(x_ref[:, j, :].astype(jnp.bfloat16), h1[...],
                        w0[...], w1[...], brz0[...],
                        bn0[0:1, :], bn0[1:2, :], H)
        nh2 = _gru_cell(s1[...], h2[...], w2[...], w3[...],
                        brz1[...], bn1[0:1, :], bn1[1:2, :], H)
        nh3 = _gru_cell(s2[...], h3[...], w4[...], w5[...],
                        brz2[...], bn2[0:1, :], bn2[1:2, :], I)
        nh4 = _gru_cell(s3[...], h4[...], w6[...], w7[...],
                        brz3[...], bn3[0:1, :], bn3[1:2, :], I)

        h1[...] = nh1
        s1[...] = nh1.astype(jnp.bfloat16)

        def _gate(body, active_from):
            if j >= active_from:
                body()
            else:
                pl.when(g >= 1)(body)

        def _w2():
            h2[...] = nh2
            s2[...] = nh2.astype(jnp.bfloat16)

        def _w3():
            h3[...] = nh3
            s3[...] = nh3.astype(jnp.bfloat16)

        def _w4():
            h4[...] = nh4

        _gate(_w2, 1)
        _gate(_w3, 2)
        _gate(_w4, 3)

        if j < 3:
            xbar_ref[:, j + CH - 3, :] = nh4
        else:
            stage[j - 3, :, :] = nh4

        if j == T % CH:
            @pl.when(g == T // CH)
            def _final():
                z = nh2
                z_ref[...] = z
                cols = []
                for k in range(K):
                    d = z - c_ref[k, :]
                    cols.append(jnp.sum(d * d, axis=1, keepdims=True))
                d2 = jnp.concatenate(cols, axis=1)
                qu = 1.0 / (1.0 + d2)
                q_ref[...] = qu / jnp.sum(qu, axis=1, keepdims=True)


def kernel(x, enc_Wih0, enc_Whh0, enc_bih0, enc_bhh0, enc_Wih1, enc_Whh1,
           enc_bih1, enc_bhh1, dec_Wih0, dec_Whh0, dec_bih0, dec_bhh0,
           dec_Wih1, dec_Whh1, dec_bih1, dec_bhh1, fc_w, fc_b, cluster):
    del fc_w, fc_b
    raw_b = (enc_bih0, enc_bhh0, enc_bih1, enc_bhh1,
             dec_bih0, dec_bhh0, dec_bih1, dec_bhh1)

    def whole(shape):
        return pl.BlockSpec(shape, lambda g: (0, 0))

    raw_w = (enc_Wih0, enc_Whh0, enc_Wih1, enc_Whh1,
             dec_Wih0, dec_Whh0, dec_Wih1, dec_Whh1)

    in_specs = [
        pl.BlockSpec((B, CH, I), lambda g: (0, jnp.minimum(g, LAST_XBLK), 0)),
        whole((K, H)),
    ]
    in_specs += [whole(w.shape) for w in raw_w]
    in_specs += [pl.BlockSpec(b.shape, lambda g: (0,)) for b in raw_b]

    out_specs = [
        pl.BlockSpec((B, CH, I),
                     lambda g: (0, jnp.minimum(jnp.maximum(g - 1, 0),
                                               LAST_XBLK), 0)),
        whole((B, H)),
        whole((B, K)),
    ]
    out_shape = [
        jax.ShapeDtypeStruct((B, T, I), jnp.float32),
        jax.ShapeDtypeStruct((B, H), jnp.float32),
        jax.ShapeDtypeStruct((B, K), jnp.float32),
    ]

    xbar, z, q = pl.pallas_call(
        _fused_kernel,
        grid=(G,),
        in_specs=in_specs,
        out_specs=out_specs,
        out_shape=out_shape,
        scratch_shapes=[
            pltpu.VMEM((B, H), jnp.float32),
            pltpu.VMEM((B, H), jnp.float32),
            pltpu.VMEM((B, I), jnp.float32),
            pltpu.VMEM((B, I), jnp.float32),
            pltpu.VMEM((B, H), jnp.bfloat16),
            pltpu.VMEM((B, H), jnp.bfloat16),
            pltpu.VMEM((B, I), jnp.bfloat16),
            pltpu.VMEM((CH - 3, B, I), jnp.float32),
            pltpu.VMEM((I, 3 * H), jnp.bfloat16),
            pltpu.VMEM((H, 3 * H), jnp.bfloat16),
            pltpu.VMEM((H, 3 * H), jnp.bfloat16),
            pltpu.VMEM((H, 3 * H), jnp.bfloat16),
            pltpu.VMEM((H, 3 * I), jnp.bfloat16),
            pltpu.VMEM((I, 3 * I), jnp.bfloat16),
            pltpu.VMEM((I, 3 * I), jnp.bfloat16),
            pltpu.VMEM((I, 3 * I), jnp.bfloat16),
            pltpu.VMEM((1, 2 * H), jnp.float32),
            pltpu.VMEM((2, H), jnp.float32),
            pltpu.VMEM((1, 2 * H), jnp.float32),
            pltpu.VMEM((2, H), jnp.float32),
            pltpu.VMEM((1, 2 * I), jnp.float32),
            pltpu.VMEM((2, I), jnp.float32),
            pltpu.VMEM((1, 2 * I), jnp.float32),
            pltpu.VMEM((2, I), jnp.float32),
        ],
        compiler_params=pltpu.CompilerParams(
            dimension_semantics=("arbitrary",),
        ),
    )(x, cluster, *raw_w, *raw_b)

    return (z, xbar, q)


# speedup vs baseline: 1.1000x; 1.1000x over previous
import jax
import jax.numpy as jnp
from jax.experimental import pallas as pl
from jax.experimental.pallas import tpu as pltpu

B, T, I, H, K = 512, 100, 128, 256, 8
CH = 8
G = (T + 3 + CH - 1) // CH + 1
LAST_XBLK = (T - 1) // CH


def _sigmoid(x):
    return 0.5 + 0.5 * jnp.tanh(0.5 * x)


def _gru_cell(x_bf, h_prev_bf, wih, whh, brz, bin_, bhn, hd):
    gi = jnp.dot(x_bf, wih, preferred_element_type=jnp.float32)
    gh = jnp.dot(h_prev_bf, whh, preferred_element_type=jnp.float32)
    rz = _sigmoid(gi[:, : 2 * hd] + gh[:, : 2 * hd] + brz)
    r = rz[:, :hd]
    zg = rz[:, hd:]
    n = jnp.tanh(gi[:, 2 * hd:] + bin_ + r * (gh[:, 2 * hd:] + bhn))
    return n + zg * (h_prev_bf.astype(jnp.float32) - n)


def _fused_kernel(x_ref, c_ref,
                  rw0, rw1, rw2, rw3, rw4, rw5, rw6, rw7,
                  bih0_ref, bhh0_ref, bih1_ref, bhh1_ref,
                  bih2_ref, bhh2_ref, bih3_ref, bhh3_ref,
                  xbar_ref, z_ref, q_ref,
                  h1, h2, h3, h4, stage,
                  w0, w1, w2, w3, w4, w5, w6, w7,
                  brz0, bn0, brz1, bn1, brz2, bn2, brz3, bn3):
    g = pl.program_id(0)

    @pl.when(g == 0)
    def _init():
        h1[...] = jnp.zeros_like(h1)
        h2[...] = jnp.zeros_like(h2)
        h3[...] = jnp.zeros_like(h3)
        h4[...] = jnp.zeros_like(h4)
        for dst, src in ((w0, rw0), (w1, rw1), (w2, rw2), (w3, rw3),
                         (w4, rw4), (w5, rw5), (w6, rw6), (w7, rw7)):
            dst[...] = jnp.swapaxes(src[...], 0, 1).astype(jnp.bfloat16)
        for brz, bn, bi, bh, hd in ((brz0, bn0, bih0_ref, bhh0_ref, H),
                                    (brz1, bn1, bih1_ref, bhh1_ref, H),
                                    (brz2, bn2, bih2_ref, bhh2_ref, I),
                                    (brz3, bn3, bih3_ref, bhh3_ref, I)):
            brz[...] = (bi[: 2 * hd] + bh[: 2 * hd]).reshape(1, 2 * hd)
            bn[...] = jnp.concatenate(
                [bi[2 * hd:].reshape(1, hd), bh[2 * hd:].reshape(1, hd)],
                axis=0)

    for p in range(CH - 3):
        xbar_ref[:, p, :] = stage[p, :, :]

    for j in range(CH):
        nh1 = _gru_cell(x_ref[:, j, :].astype(jnp.bfloat16), h1[...],
                        w0[...], w1[...], brz0[...],
                        bn0[0:1, :], bn0[1:2, :], H)
        nh2 = _gru_cell(h1[...], h2[...], w2[...], w3[...],
                        brz1[...], bn1[0:1, :], bn1[1:2, :], H)
        nh3 = _gru_cell(h2[...], h3[...], w4[...], w5[...],
                        brz2[...], bn2[0:1, :], bn2[1:2, :], I)
        nh4 = _gru_cell(h3[...], h4[...], w6[...], w7[...],
                        brz3[...], bn3[0:1, :], bn3[1:2, :], I)

        h1[...] = nh1.astype(jnp.bfloat16)

        def _gate(body, active_from):
            if j >= active_from:
                body()
            else:
                pl.when(g >= 1)(body)

        def _w2():
            h2[...] = nh2.astype(jnp.bfloat16)

        def _w3():
            h3[...] = nh3.astype(jnp.bfloat16)

        def _w4():
            h4[...] = nh4.astype(jnp.bfloat16)

        _gate(_w2, 1)
        _gate(_w3, 2)
        _gate(_w4, 3)

        if j < 3:
            xbar_ref[:, j + CH - 3, :] = nh4
        else:
            stage[j - 3, :, :] = nh4

        if j == T % CH:
            @pl.when(g == T // CH)
            def _final():
                z = nh2
                z_ref[...] = z
                cols = []
                for k in range(K):
                    d = z - c_ref[k, :]
                    cols.append(jnp.sum(d * d, axis=1, keepdims=True))
                d2 = jnp.concatenate(cols, axis=1)
                qu = 1.0 / (1.0 + d2)
                q_ref[...] = qu / jnp.sum(qu, axis=1, keepdims=True)


def kernel(x, enc_Wih0, enc_Whh0, enc_bih0, enc_bhh0, enc_Wih1, enc_Whh1,
           enc_bih1, enc_bhh1, dec_Wih0, dec_Whh0, dec_bih0, dec_bhh0,
           dec_Wih1, dec_Whh1, dec_bih1, dec_bhh1, fc_w, fc_b, cluster):
    del fc_w, fc_b
    raw_b = (enc_bih0, enc_bhh0, enc_bih1, enc_bhh1,
             dec_bih0, dec_bhh0, dec_bih1, dec_bhh1)

    def whole(shape):
        return pl.BlockSpec(shape, lambda g: (0, 0))

    raw_w = (enc_Wih0, enc_Whh0, enc_Wih1, enc_Whh1,
             dec_Wih0, dec_Whh0, dec_Wih1, dec_Whh1)

    in_specs = [
        pl.BlockSpec((B, CH, I), lambda g: (0, jnp.minimum(g, LAST_XBLK), 0)),
        whole((K, H)),
    ]
    in_specs += [whole(w.shape) for w in raw_w]
    in_specs += [pl.BlockSpec(b.shape, lambda g: (0,)) for b in raw_b]

    out_specs = [
        pl.BlockSpec((B, CH, I),
                     lambda g: (0, jnp.minimum(jnp.maximum(g - 1, 0),
                                               LAST_XBLK), 0)),
        whole((B, H)),
        whole((B, K)),
    ]
    out_shape = [
        jax.ShapeDtypeStruct((B, T, I), jnp.float32),
        jax.ShapeDtypeStruct((B, H), jnp.float32),
        jax.ShapeDtypeStruct((B, K), jnp.float32),
    ]

    xbar, z, q = pl.pallas_call(
        _fused_kernel,
        grid=(G,),
        in_specs=in_specs,
        out_specs=out_specs,
        out_shape=out_shape,
        scratch_shapes=[
            pltpu.VMEM((B, H), jnp.bfloat16),
            pltpu.VMEM((B, H), jnp.bfloat16),
            pltpu.VMEM((B, I), jnp.bfloat16),
            pltpu.VMEM((B, I), jnp.bfloat16),
            pltpu.VMEM((CH - 3, B, I), jnp.float32),
            pltpu.VMEM((I, 3 * H), jnp.bfloat16),
            pltpu.VMEM((H, 3 * H), jnp.bfloat16),
            pltpu.VMEM((H, 3 * H), jnp.bfloat16),
            pltpu.VMEM((H, 3 * H), jnp.bfloat16),
            pltpu.VMEM((H, 3 * I), jnp.bfloat16),
            pltpu.VMEM((I, 3 * I), jnp.bfloat16),
            pltpu.VMEM((I, 3 * I), jnp.bfloat16),
            pltpu.VMEM((I, 3 * I), jnp.bfloat16),
            pltpu.VMEM((1, 2 * H), jnp.float32),
            pltpu.VMEM((2, H), jnp.float32),
            pltpu.VMEM((1, 2 * H), jnp.float32),
            pltpu.VMEM((2, H), jnp.float32),
            pltpu.VMEM((1, 2 * I), jnp.float32),
            pltpu.VMEM((2, I), jnp.float32),
            pltpu.VMEM((1, 2 * I), jnp.float32),
            pltpu.VMEM((2, I), jnp.float32),
        ],
        compiler_params=pltpu.CompilerParams(
            dimension_semantics=("arbitrary",),
        ),
    )(x, cluster, *raw_w, *raw_b)

    return (z, xbar, q)


# speedup vs baseline: 1.1043x; 1.0039x over previous
import jax
import jax.numpy as jnp
from jax.experimental import pallas as pl
from jax.experimental.pallas import tpu as pltpu

B, T, I, H, K = 512, 100, 128, 256, 8
CH = 8
G = (T + 3 + CH - 1) // CH + 1
LAST_XBLK = (T - 1) // CH


def _sigmoid(x):
    return 0.5 + 0.5 * jnp.tanh(0.5 * x)


def _gru_cell(x_bf, h_prev, wih, whh, brz, bin_, bhn, hd):
    gi = jnp.dot(x_bf, wih, preferred_element_type=jnp.float32)
    gh = jnp.dot(h_prev.astype(jnp.bfloat16), whh,
                 preferred_element_type=jnp.float32)
    rz = _sigmoid(gi[:, : 2 * hd] + gh[:, : 2 * hd] + brz)
    r = rz[:, :hd]
    zg = rz[:, hd:]
    n = jnp.tanh(gi[:, 2 * hd:] + bin_ + r * (gh[:, 2 * hd:] + bhn))
    return n + zg * (h_prev - n)


def _fused_kernel(x_ref, c_ref,
                  rw0, rw1, rw2, rw3, rw4, rw5, rw6, rw7,
                  bih0_ref, bhh0_ref, bih1_ref, bhh1_ref,
                  bih2_ref, bhh2_ref, bih3_ref, bhh3_ref,
                  xbar_ref, z_ref, q_ref,
                  h1, h2, h3, h4, s1, s2, s3, stage,
                  w0, w1, w2, w3, w4, w5, w6, w7,
                  brz0, bn0, brz1, bn1, brz2, bn2, brz3, bn3):
    g = pl.program_id(0)

    @pl.when(g == 0)
    def _init():
        h1[...] = jnp.zeros_like(h1)
        h2[...] = jnp.zeros_like(h2)
        h3[...] = jnp.zeros_like(h3)
        h4[...] = jnp.zeros_like(h4)
        s1[...] = jnp.zeros_like(s1)
        s2[...] = jnp.zeros_like(s2)
        s3[...] = jnp.zeros_like(s3)
        for dst, src in ((w0, rw0), (w1, rw1), (w2, rw2), (w3, rw3),
                         (w4, rw4), (w5, rw5), (w6, rw6), (w7, rw7)):
            dst[...] = jnp.swapaxes(src[...], 0, 1).astype(jnp.bfloat16)
        for brz, bn, bi, bh, hd in ((brz0, bn0, bih0_ref, bhh0_ref, H),
                                    (brz1, bn1, bih1_ref, bhh1_ref, H),
                                    (brz2, bn2, bih2_ref, bhh2_ref, I),
                                    (brz3, bn3, bih3_ref, bhh3_ref, I)):
            brz[...] = (bi[: 2 * hd] + bh[: 2 * hd]).reshape(1, 2 * hd)
            bn[...] = jnp.concatenate(
                [bi[2 * hd:].reshape(1, hd), bh[2 * hd:].reshape(1, hd)],
                axis=0)

    for p in range(CH - 3):
        xbar_ref[:, p, :] = stage[p, :, :]

    for j in range(CH):
        nh1 = _gru_cell(x_ref[:, j, :].astype(jnp.bfloat16), h1[...],
                        w0[...], w1[...], brz0[...],
                        bn0[0:1, :], bn0[1:2, :], H)
        nh2 = _gru_cell(s1[...], h2[...], w2[...], w3[...],
                        brz1[...], bn1[0:1, :], bn1[1:2, :], H)
        nh3 = _gru_cell(s2[...], h3[...], w4[...], w5[...],
                        brz2[...], bn2[0:1, :], bn2[1:2, :], I)
        nh4 = _gru_cell(s3[...], h4[...], w6[...], w7[...],
                        brz3[...], bn3[0:1, :], bn3[1:2, :], I)

        h1[...] = nh1
        s1[...] = nh1.astype(jnp.bfloat16)

        def _gate(body, active_from):
            if j >= active_from:
                body()
            else:
                pl.when(g >= 1)(body)

        def _w2():
            h2[...] = nh2
            s2[...] = nh2.astype(jnp.bfloat16)

        def _w3():
            h3[...] = nh3
            s3[...] = nh3.astype(jnp.bfloat16)

        def _w4():
            h4[...] = nh4

        _gate(_w2, 1)
        _gate(_w3, 2)
        _gate(_w4, 3)

        if j < 3:
            xbar_ref[:, j + CH - 3, :] = nh4
        else:
            stage[j - 3, :, :] = nh4

        if j == T % CH:
            @pl.when(g == T // CH)
            def _final():
                z = nh2
                z_ref[...] = z
                cols = []
                for k in range(K):
                    d = z - c_ref[k, :]
                    cols.append(jnp.sum(d * d, axis=1, keepdims=True))
                d2 = jnp.concatenate(cols, axis=1)
                qu = 1.0 / (1.0 + d2)
                q_ref[...] = qu / jnp.sum(qu, axis=1, keepdims=True)


def kernel(x, enc_Wih0, enc_Whh0, enc_bih0, enc_bhh0, enc_Wih1, enc_Whh1,
           enc_bih1, enc_bhh1, dec_Wih0, dec_Whh0, dec_bih0, dec_bhh0,
           dec_Wih1, dec_Whh1, dec_bih1, dec_bhh1, fc_w, fc_b, cluster):
    del fc_w, fc_b
    raw_b = (enc_bih0, enc_bhh0, enc_bih1, enc_bhh1,
             dec_bih0, dec_bhh0, dec_bih1, dec_bhh1)

    def whole(shape):
        return pl.BlockSpec(shape, lambda g: (0, 0))

    raw_w = (enc_Wih0, enc_Whh0, enc_Wih1, enc_Whh1,
             dec_Wih0, dec_Whh0, dec_Wih1, dec_Whh1)

    in_specs = [
        pl.BlockSpec((B, CH, I), lambda g: (0, jnp.minimum(g, LAST_XBLK), 0)),
        whole((K, H)),
    ]
    in_specs += [whole(w.shape) for w in raw_w]
    in_specs += [pl.BlockSpec(b.shape, lambda g: (0,)) for b in raw_b]

    out_specs = [
        pl.BlockSpec((B, CH, I),
                     lambda g: (0, jnp.minimum(jnp.maximum(g - 1, 0),
                                               LAST_XBLK), 0)),
        whole((B, H)),
        whole((B, K)),
    ]
    out_shape = [
        jax.ShapeDtypeStruct((B, T, I), jnp.float32),
        jax.ShapeDtypeStruct((B, H), jnp.float32),
        jax.ShapeDtypeStruct((B, K), jnp.float32),
    ]

    xbar, z, q = pl.pallas_call(
        _fused_kernel,
        grid=(G,),
        in_specs=in_specs,
        out_specs=out_specs,
        out_shape=out_shape,
        scratch_shapes=[
            pltpu.VMEM((B, H), jnp.float32),
            pltpu.VMEM((B, H), jnp.float32),
            pltpu.VMEM((B, I), jnp.float32),
            pltpu.VMEM((B, I), jnp.float32),
            pltpu.VMEM((B, H), jnp.bfloat16),
            pltpu.VMEM((B, H), jnp.bfloat16),
            pltpu.VMEM((B, I), jnp.bfloat16),
            pltpu.VMEM((CH - 3, B, I), jnp.float32),
            pltpu.VMEM((I, 3 * H), jnp.bfloat16),
            pltpu.VMEM((H, 3 * H), jnp.bfloat16),
            pltpu.VMEM((H, 3 * H), jnp.bfloat16),
            pltpu.VMEM((H, 3 * H), jnp.bfloat16),
            pltpu.VMEM((H, 3 * I), jnp.bfloat16),
            pltpu.VMEM((I, 3 * I), jnp.bfloat16),
            pltpu.VMEM((I, 3 * I), jnp.bfloat16),
            pltpu.VMEM((I, 3 * I), jnp.bfloat16),
            pltpu.VMEM((1, 2 * H), jnp.float32),
            pltpu.VMEM((2, H), jnp.float32),
            pltpu.VMEM((1, 2 * H), jnp.float32),
            pltpu.VMEM((2, H), jnp.float32),
            pltpu.VMEM((1, 2 * I), jnp.float32),
            pltpu.VMEM((2, I), jnp.float32),
            pltpu.VMEM((1, 2 * I), jnp.float32),
            pltpu.VMEM((2, I), jnp.float32),
        ],
        compiler_params=pltpu.CompilerParams(
            dimension_semantics=("arbitrary",),
        ),
    )(x, cluster, *raw_w, *raw_b)

    return (z, xbar, q)
